# baseline (speedup 1.0000x reference)
import functools

import jax
import jax.numpy as jnp
from jax import lax
from jax.experimental import pallas as pl
from jax.experimental.pallas import tpu as pltpu
from jax.experimental.pallas import tpu_sc as plsc

N_NODES = 10000
N_PAD = 10240
D_FEAT = 128
N_EDGES = 320000
NUM_LAYERS = 3
CHUNK = 64
_DO_SCATTER = False
_DO_MUL = False


def _scatter_body(nc, ns, chunks_per_tile, rows_per_tile,
                  h, srcr, dstr, wr, zrows, out,
                  idx_s, idx_d, w_v, rows, acc, sem_i, sem_g, sem_s):
  c = lax.axis_index("c")
  s = lax.axis_index("s")
  wid = c * ns + s
  ch = chunks_per_tile
  base = wid * ch

  pltpu.sync_copy(zrows, acc.at[pl.ds(s * rows_per_tile, rows_per_tile)])
  plsc.subcore_barrier()

  def idx_issue(kk, slot):
    off = (base + kk) * CHUNK
    pltpu.async_copy(srcr.at[pl.ds(off, CHUNK)], idx_s.at[slot],
                     sem_i.at[slot])
    pltpu.async_copy(dstr.at[pl.ds(off, CHUNK)], idx_d.at[slot],
                     sem_i.at[slot])
    pltpu.async_copy(wr.at[pl.ds(off, CHUNK)], w_v.at[slot],
                     sem_i.at[slot])

  def idx_wait(kk, slot):
    off = (base + kk) * CHUNK
    pltpu.make_async_copy(srcr.at[pl.ds(off, CHUNK)], idx_s.at[slot],
                          sem_i.at[slot]).wait()
    pltpu.make_async_copy(dstr.at[pl.ds(off, CHUNK)], idx_d.at[slot],
                          sem_i.at[slot]).wait()
    pltpu.make_async_copy(wr.at[pl.ds(off, CHUNK)], w_v.at[slot],
                          sem_i.at[slot]).wait()

  def gather_issue(slot, buf):
    pltpu.async_copy(h.at[idx_s.at[slot]], rows.at[buf], sem_g.at[buf])

  def gather_wait(slot, buf):
    pltpu.make_async_copy(h.at[idx_s.at[slot]], rows.at[buf],
                          sem_g.at[buf]).wait()

  def scat_issue(slot, buf):
    pltpu.async_copy(rows.at[buf], acc.at[idx_d.at[slot]], sem_s.at[buf],
                     add=True)

  def scat_wait(slot, buf):
    pltpu.make_async_copy(rows.at[buf], acc.at[idx_d.at[slot]],
                          sem_s.at[buf]).wait()

  def mul(slot, buf):
    rb = rows.at[buf]

    def mul_group(g, carry2):
      wvec = w_v[slot, pl.ds(g * 16, 16)]
      for i in range(16):
        wgt = wvec[i]
        r = g * 16 + i
        for j in range(D_FEAT // 16):
          rb[r, pl.ds(j * 16, 16)] = rb[r, pl.ds(j * 16, 16)] * wgt
      return carry2

    if _DO_MUL:
      lax.fori_loop(0, CHUNK // 16, mul_group, 0)

  for kk in range(4):
    idx_issue(kk, kk)
  idx_wait(0, 0)
  gather_issue(0, 0)
  idx_wait(1, 1)
  gather_issue(1, 1)

  def blk(i8, carry):
    k0 = i8 * 8
    for b in range(8):
      k = k0 + b
      rb = b % 4
      s8 = b

      @pl.when(k >= 2)
      def _():
        if _DO_SCATTER:
          scat_wait((b + 6) % 8, (b + 2) % 4)

      @pl.when(k + 2 < ch)
      def _():
        idx_wait(k + 2, (b + 2) % 8)
        gather_issue((b + 2) % 8, (b + 2) % 4)

      @pl.when(k + 4 < ch)
      def _():
        idx_issue(k + 4, (b + 4) % 8)

      gather_wait(s8, rb)
      mul(s8, rb)
      if _DO_SCATTER:
        scat_issue(s8, rb)
    return carry

  lax.fori_loop(0, ch // 8, blk, 0)
  if _DO_SCATTER:
    scat_wait((ch - 2) % 8, (ch - 2) % 4)
    scat_wait((ch - 1) % 8, (ch - 1) % 4)

  plsc.subcore_barrier()
  r0 = s * rows_per_tile
  pltpu.sync_copy(acc.at[pl.ds(r0, rows_per_tile)],
                  out.at[c, pl.ds(r0, rows_per_tile)])


def _make_scatter(nc, ns):
  nw = nc * ns
  grain = nw * CHUNK * 8
  epad = ((N_EDGES + grain - 1) // grain) * grain
  chunks_per_tile = epad // (nw * CHUNK)
  rows_per_tile = N_PAD // ns
  body = functools.partial(_scatter_body, nc, ns, chunks_per_tile,
                           rows_per_tile)
  call = pl.kernel(
      body,
      out_type=jax.ShapeDtypeStruct((nc, N_PAD, D_FEAT), jnp.float32),
      mesh=plsc.VectorSubcoreMesh(core_axis_name="c", subcore_axis_name="s"),
      compiler_params=pltpu.CompilerParams(use_tc_tiling_on_sc=False),
      scratch_types=[
          pltpu.VMEM((8, CHUNK), jnp.int32),
          pltpu.VMEM((8, CHUNK), jnp.int32),
          pltpu.VMEM((8, CHUNK), jnp.float32),
          pltpu.VMEM((4, CHUNK, D_FEAT), jnp.float32),
          pltpu.VMEM_SHARED((N_PAD, D_FEAT), jnp.float32),
          pltpu.SemaphoreType.DMA((8,)),
          pltpu.SemaphoreType.DMA((4,)),
          pltpu.SemaphoreType.DMA((4,)),
      ],
  )
  return call, epad, rows_per_tile


def _merge_mid_body(p_ref, accp_ref, h_ref, accn_ref):
  hsum = p_ref[0] + p_ref[1]
  h_ref[...] = hsum
  accn_ref[...] = accp_ref[...] + hsum


def _merge_final_body(p_ref, accp_ref, o_ref):
  o_ref[...] = (1.0 / (NUM_LAYERS + 1)) * (
      accp_ref[...] + p_ref[0] + p_ref[1])


_ROWS_BLK = 1024


def _merge_mid(partials, acc_prev):
  grid = N_PAD // _ROWS_BLK
  return pl.pallas_call(
      _merge_mid_body,
      grid=(grid,),
      in_specs=[
          pl.BlockSpec((2, _ROWS_BLK, D_FEAT), lambda i: (0, i, 0)),
          pl.BlockSpec((_ROWS_BLK, D_FEAT), lambda i: (i, 0)),
      ],
      out_specs=[
          pl.BlockSpec((_ROWS_BLK, D_FEAT), lambda i: (i, 0)),
          pl.BlockSpec((_ROWS_BLK, D_FEAT), lambda i: (i, 0)),
      ],
      out_shape=[
          jax.ShapeDtypeStruct((N_PAD, D_FEAT), jnp.float32),
          jax.ShapeDtypeStruct((N_PAD, D_FEAT), jnp.float32),
      ],
  )(partials, acc_prev)


def _merge_final(partials, acc_prev):
  grid = N_PAD // _ROWS_BLK
  return pl.pallas_call(
      _merge_final_body,
      grid=(grid,),
      in_specs=[
          pl.BlockSpec((2, _ROWS_BLK, D_FEAT), lambda i: (0, i, 0)),
          pl.BlockSpec((_ROWS_BLK, D_FEAT), lambda i: (i, 0)),
      ],
      out_specs=pl.BlockSpec((_ROWS_BLK, D_FEAT), lambda i: (i, 0)),
      out_shape=jax.ShapeDtypeStruct((N_PAD, D_FEAT), jnp.float32),
  )(partials, acc_prev)


def kernel(x, edge_index, edge_weight):
  info = plsc.get_sparse_core_info()
  nc, ns = info.num_cores, info.num_subcores
  scatter, epad, rows_per_tile = _make_scatter(nc, ns)

  pad = epad - N_EDGES
  src = jnp.concatenate(
      [edge_index[0].astype(jnp.int32), jnp.zeros((pad,), jnp.int32)])
  dst = jnp.concatenate(
      [edge_index[1].astype(jnp.int32), jnp.zeros((pad,), jnp.int32)])
  w = jnp.concatenate(
      [edge_weight.astype(jnp.float32), jnp.zeros((pad,), jnp.float32)])
  zrows = jnp.zeros((rows_per_tile, D_FEAT), jnp.float32)

  xp = jnp.concatenate(
      [x, jnp.zeros((N_PAD - N_NODES, D_FEAT), jnp.float32)])
  h = xp
  acc = xp
  for layer in range(NUM_LAYERS):
    partials = scatter(h, src, dst, w, zrows)
    if layer < NUM_LAYERS - 1:
      h, acc = _merge_mid(partials, acc)
    else:
      out = _merge_final(partials, acc)
  return out[:N_NODES]


# speedup vs baseline: 1.3038x; 1.3038x over previous
import functools

import jax
import jax.numpy as jnp
from jax import lax
from jax.experimental import pallas as pl
from jax.experimental.pallas import tpu as pltpu
from jax.experimental.pallas import tpu_sc as plsc

N_NODES = 10000
N_PAD = 10240
D_FEAT = 128
D_PK = D_FEAT // 2
N_EDGES = 320000
NUM_LAYERS = 3
CHUNK = 64


def _scatter_body(nc, ns, chunks_per_tile, rows_per_tile,
                  h, srcr, dstr, wr, zrows, out,
                  idx_s, idx_d, w_v, prows, frows, acc, sem_i, sem_g, sem_s):
  c = lax.axis_index("c")
  s = lax.axis_index("s")
  wid = c * ns + s
  ch = chunks_per_tile
  base = wid * ch

  pltpu.sync_copy(zrows, acc.at[pl.ds(s * rows_per_tile, rows_per_tile)])
  plsc.subcore_barrier()

  def idx_issue(kk, slot):
    off = (base + kk) * CHUNK
    pltpu.async_copy(srcr.at[pl.ds(off, CHUNK)], idx_s.at[slot],
                     sem_i.at[slot])
    pltpu.async_copy(dstr.at[pl.ds(off, CHUNK)], idx_d.at[slot],
                     sem_i.at[slot])
    pltpu.async_copy(wr.at[pl.ds(off, CHUNK)], w_v.at[slot],
                     sem_i.at[slot])

  def idx_wait(kk, slot):
    off = (base + kk) * CHUNK
    pltpu.make_async_copy(srcr.at[pl.ds(off, CHUNK)], idx_s.at[slot],
                          sem_i.at[slot]).wait()
    pltpu.make_async_copy(dstr.at[pl.ds(off, CHUNK)], idx_d.at[slot],
                          sem_i.at[slot]).wait()
    pltpu.make_async_copy(wr.at[pl.ds(off, CHUNK)], w_v.at[slot],
                          sem_i.at[slot]).wait()

  def gather_issue(slot, buf):
    pltpu.async_copy(h.at[idx_s.at[slot]], prows.at[buf], sem_g.at[buf])

  def gather_wait(slot, buf):
    pltpu.make_async_copy(h.at[idx_s.at[slot]], prows.at[buf],
                          sem_g.at[buf]).wait()

  def scat_issue(slot, buf):
    pltpu.async_copy(frows.at[buf], acc.at[idx_d.at[slot]], sem_s.at[buf],
                     add=True)

  def scat_wait(slot, buf):
    pltpu.make_async_copy(frows.at[buf], acc.at[idx_d.at[slot]],
                          sem_s.at[buf]).wait()

  himask = jnp.full((16,), -65536, dtype=jnp.int32)
  shamt = jnp.full((16,), 16, dtype=jnp.int32)

  def mul(slot, pbuf, fbuf):
    pb = prows.at[pbuf]
    fb = frows.at[fbuf]

    def mul_group(g, carry2):
      wvec = w_v[slot, pl.ds(g * 16, 16)]
      for i in range(16):
        wgt = wvec[i]
        r = g * 16 + i
        for j in range(D_PK // 16):
          wrd = pb[r, pl.ds(j * 16, 16)]
          flo = lax.bitcast_convert_type(lax.shift_left(wrd, shamt), jnp.float32)
          fhi = lax.bitcast_convert_type(lax.bitwise_and(wrd, himask), jnp.float32)
          fb[r, pl.ds(j * 32, 16)] = flo * wgt
          fb[r, pl.ds(j * 32 + 16, 16)] = fhi * wgt
      return carry2

    lax.fori_loop(0, CHUNK // 16, mul_group, 0)

  for kk in range(4):
    idx_issue(kk, kk)
  idx_wait(0, 0)
  gather_issue(0, 0)
  idx_wait(1, 1)
  gather_issue(1, 1)

  def blk(i8, carry):
    k0 = i8 * 8
    for b in range(8):
      k = k0 + b

      @pl.when(k >= 2)
      def _():
        scat_wait((b + 6) % 8, b % 2)

      @pl.when(k + 2 < ch)
      def _():
        idx_wait(k + 2, (b + 2) % 8)
        gather_issue((b + 2) % 8, (b + 2) % 4)

      @pl.when(k + 4 < ch)
      def _():
        idx_issue(k + 4, (b + 4) % 8)

      gather_wait(b, b % 4)
      mul(b, b % 4, b % 2)
      scat_issue(b, b % 2)
    return carry

  lax.fori_loop(0, ch // 8, blk, 0)
  scat_wait((ch - 2) % 8, (ch - 2) % 2)
  scat_wait((ch - 1) % 8, (ch - 1) % 2)

  plsc.subcore_barrier()
  r0 = s * rows_per_tile
  pltpu.sync_copy(acc.at[pl.ds(r0, rows_per_tile)],
                  out.at[c, pl.ds(r0, rows_per_tile)])


def _make_scatter(nc, ns):
  nw = nc * ns
  grain = nw * CHUNK * 8
  epad = ((N_EDGES + grain - 1) // grain) * grain
  chunks_per_tile = epad // (nw * CHUNK)
  rows_per_tile = N_PAD // ns
  body = functools.partial(_scatter_body, nc, ns, chunks_per_tile,
                           rows_per_tile)
  call = pl.kernel(
      body,
      out_type=jax.ShapeDtypeStruct((nc, N_PAD, D_FEAT), jnp.float32),
      mesh=plsc.VectorSubcoreMesh(core_axis_name="c", subcore_axis_name="s"),
      compiler_params=pltpu.CompilerParams(use_tc_tiling_on_sc=False),
      scratch_types=[
          pltpu.VMEM((8, CHUNK), jnp.int32),
          pltpu.VMEM((8, CHUNK), jnp.int32),
          pltpu.VMEM((8, CHUNK), jnp.float32),
          pltpu.VMEM((4, CHUNK, D_PK), jnp.int32),
          pltpu.VMEM((2, CHUNK, D_FEAT), jnp.float32),
          pltpu.VMEM_SHARED((N_PAD, D_FEAT), jnp.float32),
          pltpu.SemaphoreType.DMA((8,)),
          pltpu.SemaphoreType.DMA((4,)),
          pltpu.SemaphoreType.DMA((2,)),
      ],
  )
  return call, epad, rows_per_tile


def _pack_bf16(hf32):
  hb = lax.bitcast_convert_type(hf32.astype(jnp.bfloat16), jnp.uint16)
  v = hb.reshape(hf32.shape[0], D_FEAT // 32, 2, 16).astype(jnp.uint32)
  packed = v[:, :, 0, :] | (v[:, :, 1, :] << 16)
  return lax.bitcast_convert_type(packed, jnp.int32).reshape(-1, D_PK)


def _merge_mid_body(p_ref, accp_ref, h_ref, accn_ref):
  hsum = p_ref[0] + p_ref[1]
  h_ref[...] = hsum
  accn_ref[...] = accp_ref[...] + hsum


def _merge_final_body(p_ref, accp_ref, o_ref):
  o_ref[...] = (1.0 / (NUM_LAYERS + 1)) * (
      accp_ref[...] + p_ref[0] + p_ref[1])


_ROWS_BLK = 1024


def _merge_mid(partials, acc_prev):
  grid = N_PAD // _ROWS_BLK
  return pl.pallas_call(
      _merge_mid_body,
      grid=(grid,),
      in_specs=[
          pl.BlockSpec((2, _ROWS_BLK, D_FEAT), lambda i: (0, i, 0)),
          pl.BlockSpec((_ROWS_BLK, D_FEAT), lambda i: (i, 0)),
      ],
      out_specs=[
          pl.BlockSpec((_ROWS_BLK, D_FEAT), lambda i: (i, 0)),
          pl.BlockSpec((_ROWS_BLK, D_FEAT), lambda i: (i, 0)),
      ],
      out_shape=[
          jax.ShapeDtypeStruct((N_PAD, D_FEAT), jnp.float32),
          jax.ShapeDtypeStruct((N_PAD, D_FEAT), jnp.float32),
      ],
  )(partials, acc_prev)


def _merge_final(partials, acc_prev):
  grid = N_PAD // _ROWS_BLK
  return pl.pallas_call(
      _merge_final_body,
      grid=(grid,),
      in_specs=[
          pl.BlockSpec((2, _ROWS_BLK, D_FEAT), lambda i: (0, i, 0)),
          pl.BlockSpec((_ROWS_BLK, D_FEAT), lambda i: (i, 0)),
      ],
      out_specs=pl.BlockSpec((_ROWS_BLK, D_FEAT), lambda i: (i, 0)),
      out_shape=jax.ShapeDtypeStruct((N_PAD, D_FEAT), jnp.float32),
  )(partials, acc_prev)


def kernel(x, edge_index, edge_weight):
  info = plsc.get_sparse_core_info()
  nc, ns = info.num_cores, info.num_subcores
  scatter, epad, rows_per_tile = _make_scatter(nc, ns)

  pad = epad - N_EDGES
  src = jnp.concatenate(
      [edge_index[0].astype(jnp.int32), jnp.zeros((pad,), jnp.int32)])
  dst = jnp.concatenate(
      [edge_index[1].astype(jnp.int32), jnp.zeros((pad,), jnp.int32)])
  w = jnp.concatenate(
      [edge_weight.astype(jnp.float32), jnp.zeros((pad,), jnp.float32)])
  zrows = jnp.zeros((rows_per_tile, D_FEAT), jnp.float32)

  xp = jnp.concatenate(
      [x, jnp.zeros((N_PAD - N_NODES, D_FEAT), jnp.float32)])
  h = xp
  acc = xp
  for layer in range(NUM_LAYERS):
    partials = scatter(_pack_bf16(h), src, dst, w, zrows)
    if layer < NUM_LAYERS - 1:
      h, acc = _merge_mid(partials, acc)
    else:
      out = _merge_final(partials, acc)
  return out[:N_NODES]
